# baseline (device time: 39842 ns/iter reference)
import jax
import jax.numpy as jnp
from jax import lax
from jax.experimental import pallas as pl
from jax.experimental.pallas import tpu as pltpu

N_DEV = 32


def kernel(x, w_mat):
    m_per, k = x.shape
    _, n = w_mat.shape
    n_per = n // N_DEV
    m_total = m_per * N_DEV

    def body(x_ref, w_ref, out_ref,
             send_buf, recv_buf, amax_send, amax_recv,
             send_sems, recv_sems, asend_sems, arecv_sems):
        me = lax.axis_index("i")

        y = jnp.dot(x_ref[...], w_ref[...],
                    preferred_element_type=jnp.float32,
                    precision=lax.Precision.DEFAULT)
        y = jnp.maximum(y, 0.0)
        amax_send[...] = jnp.full((8, 128), jnp.max(y), jnp.float32)

        barrier_sem = pltpu.get_barrier_semaphore()
        for d in range(N_DEV):
            pl.semaphore_signal(
                barrier_sem, inc=1,
                device_id=(d,), device_id_type=pl.DeviceIdType.MESH,
            )
        pl.semaphore_wait(barrier_sem, N_DEV)

        for d in range(N_DEV):
            pltpu.make_async_remote_copy(
                src_ref=amax_send,
                dst_ref=amax_recv.at[me],
                send_sem=asend_sems.at[d],
                recv_sem=arecv_sems.at[me],
                device_id=(d,),
                device_id_type=pl.DeviceIdType.MESH,
            ).start()
        for d in range(N_DEV):
            pltpu.make_async_remote_copy(
                src_ref=amax_send, dst_ref=amax_send,
                send_sem=asend_sems.at[d], recv_sem=arecv_sems.at[0],
                device_id=(0,), device_id_type=pl.DeviceIdType.MESH,
            ).wait_send()
        for j in range(N_DEV):
            pltpu.make_async_remote_copy(
                src_ref=amax_recv.at[j], dst_ref=amax_recv.at[j],
                send_sem=asend_sems.at[0], recv_sem=arecv_sems.at[j],
                device_id=(0,), device_id_type=pl.DeviceIdType.MESH,
            ).wait_recv()

        amax = jnp.max(amax_recv[...])
        scale = amax / 448.0
        inv = jnp.where(scale > 0.0, 1.0 / scale, 0.0)
        yq = jnp.minimum(y * inv, 448.0).astype(jnp.float8_e4m3fn)
        for d in range(N_DEV):
            send_buf[d] = yq[:, d * n_per:(d + 1) * n_per]

        for d in range(N_DEV):
            pltpu.make_async_remote_copy(
                src_ref=send_buf.at[d],
                dst_ref=recv_buf.at[me],
                send_sem=send_sems.at[d],
                recv_sem=recv_sems.at[me],
                device_id=(d,),
                device_id_type=pl.DeviceIdType.MESH,
            ).start()
        for d in range(N_DEV):
            pltpu.make_async_remote_copy(
                src_ref=send_buf.at[d], dst_ref=send_buf.at[d],
                send_sem=send_sems.at[d], recv_sem=recv_sems.at[0],
                device_id=(0,), device_id_type=pl.DeviceIdType.MESH,
            ).wait_send()
        for j in range(N_DEV):
            pltpu.make_async_remote_copy(
                src_ref=recv_buf.at[j], dst_ref=recv_buf.at[j],
                send_sem=send_sems.at[0], recv_sem=recv_sems.at[j],
                device_id=(0,), device_id_type=pl.DeviceIdType.MESH,
            ).wait_recv()

        assembled = recv_buf[...].astype(jnp.float32).reshape(m_total, n_per)
        out_ref[...] = assembled * scale

    return pl.pallas_call(
        body,
        out_shape=jax.ShapeDtypeStruct((m_total, n_per), jnp.float32),
        in_specs=[
            pl.BlockSpec(memory_space=pltpu.VMEM),
            pl.BlockSpec(memory_space=pltpu.VMEM),
        ],
        out_specs=pl.BlockSpec(memory_space=pltpu.VMEM),
        scratch_shapes=[
            pltpu.VMEM((N_DEV, m_per, n_per), jnp.float8_e4m3fn),
            pltpu.VMEM((N_DEV, m_per, n_per), jnp.float8_e4m3fn),
            pltpu.VMEM((8, 128), jnp.float32),
            pltpu.VMEM((N_DEV, 8, 128), jnp.float32),
            pltpu.SemaphoreType.DMA((N_DEV,)),
            pltpu.SemaphoreType.DMA((N_DEV,)),
            pltpu.SemaphoreType.DMA((N_DEV,)),
            pltpu.SemaphoreType.DMA((N_DEV,)),
        ],
        compiler_params=pltpu.CompilerParams(
            vmem_limit_bytes=100 * 1024 * 1024,
            collective_id=0,
        ),
    )(x, w_mat)
